# baseline (speedup 1.0000x reference)
import functools
import jax
import jax.numpy as jnp
from jax import lax
from jax.experimental import pallas as pl
from jax.experimental.pallas import tpu as pltpu
from jax.experimental.pallas import tpu_sc as plsc


def _make_sc_gather(N, D, n_per, CH):
    info = plsc.get_sparse_core_info()
    NC = info.num_cores

    mesh = plsc.VectorSubcoreMesh(core_axis_name="c", subcore_axis_name="s")

    @functools.partial(
        pl.kernel,
        mesh=mesh,
        out_type=jax.ShapeDtypeStruct((N, 2 * D), jnp.float32),
        scratch_types=[
            pltpu.VMEM((n_per,), jnp.int32),
            pltpu.VMEM((n_per,), jnp.int32),
            pltpu.VMEM((3, CH, D), jnp.float32),
            pltpu.VMEM((3, CH, D), jnp.float32),
            pltpu.SemaphoreType.DMA,
            pltpu.SemaphoreType.DMA,
        ],
    )
    def gather_kernel(table_hbm, idx_hbm, out_hbm, idx1_v, idx2_v, buf1_v,
                      buf2_v, gsem, wsem):
        c = lax.axis_index("c")
        s = lax.axis_index("s")
        wid = s * NC + c
        base = wid * n_per
        pltpu.sync_copy(idx_hbm.at[0, pl.ds(base, n_per)], idx1_v)
        pltpu.sync_copy(idx_hbm.at[1, pl.ds(base, n_per)], idx2_v)

        nch = n_per // CH
        S = 3
        gath = [None] * S
        writes = [None] * S

        def start_gather(k):
            sl = k % S
            off = k * CH
            g1 = pltpu.async_copy(
                table_hbm.at[idx1_v.at[pl.ds(off, CH)]], buf1_v.at[sl], gsem
            )
            g2 = pltpu.async_copy(
                table_hbm.at[idx2_v.at[pl.ds(off, CH)]], buf2_v.at[sl], gsem
            )
            gath[sl] = (g1, g2)

        def start_write(k):
            sl = k % S
            off = k * CH
            gath[sl][0].wait()
            gath[sl][1].wait()
            w1 = pltpu.async_copy(
                buf1_v.at[sl], out_hbm.at[pl.ds(base + off, CH), pl.ds(0, D)],
                wsem,
            )
            w2 = pltpu.async_copy(
                buf2_v.at[sl], out_hbm.at[pl.ds(base + off, CH), pl.ds(D, D)],
                wsem,
            )
            writes[sl] = (w1, w2)

        for k in range(nch):
            sl = k % S
            if writes[sl] is not None:
                writes[sl][0].wait()
                writes[sl][1].wait()
                writes[sl] = None
            start_gather(k)
            if k >= 2:
                start_write(k - 2)
        for k in range(max(0, nch - 2), nch):
            start_write(k)
        for sl in range(S):
            if writes[sl] is not None:
                writes[sl][0].wait()
                writes[sl][1].wait()

    return gather_kernel


_IB = 64


def _mlp_body(g_ref, w1_ref, b1_ref, w2_ref, b2_ref, i_ref, out_ref):
    H = i_ref.shape[0]
    f = g_ref[0, 0]
    t = jnp.dot(
        f.astype(jnp.bfloat16), w1_ref[...].astype(jnp.bfloat16),
        preferred_element_type=jnp.float32,
    ) + b1_ref[...]
    h = jnp.maximum(t, 0.0)
    o = jnp.dot(h, w2_ref[...], preferred_element_type=jnp.float32)
    rows = [
        lax.dot_general(
            o[r * H:(r + 1) * H], i_ref[...], (((0,), (0,)), ((), ())),
            preferred_element_type=jnp.float32,
        )
        for r in range(_IB)
    ]
    out_ref[0] = jnp.concatenate(rows, axis=0) + b2_ref[...]


def _mlp_call(gathered, W1, b1r, W2, b2row, I, B, H, D, interpret=False):
    n = _IB * H
    grid = (B, H // _IB)
    return pl.pallas_call(
        _mlp_body,
        grid=grid,
        in_specs=[
            pl.BlockSpec((1, 1, n, 2 * D), lambda b, i: (b, i, 0, 0)),
            pl.BlockSpec((2 * D, 64), lambda b, i: (0, 0)),
            pl.BlockSpec((1, 64), lambda b, i: (0, 0)),
            pl.BlockSpec((64, 1), lambda b, i: (0, 0)),
            pl.BlockSpec((1, H), lambda b, i: (0, 0)),
            pl.BlockSpec((H, H), lambda b, i: (0, 0)),
        ],
        out_specs=pl.BlockSpec((1, _IB, H), lambda b, i: (b, i, 0)),
        out_shape=jax.ShapeDtypeStruct((B, H, H), jnp.float32),
        interpret=interpret,
    )(gathered.reshape(B, H // _IB, n, 2 * D), W1, b1r, W2, b2row, I)


def kernel(pair_repr, helix_ends_list, W1, b1, W2, b2):
    B, L, _, D = pair_repr.shape
    H = helix_ends_list.shape[1]
    i5 = helix_ends_list[:, :, 1]
    i3 = helix_ends_list[:, :, 2]

    boff = (jnp.arange(B, dtype=jnp.int32) * (L * L))[:, None, None]
    idx1 = boff + i5[:, :, None] * L + i5[:, None, :]
    idx2 = boff + i3[:, :, None] * L + i3[:, None, :]
    idx_all = jnp.stack([idx1.reshape(-1), idx2.reshape(-1)], axis=0)

    table = pair_repr.reshape(B * L * L, D)
    N = B * H * H
    n_per = N // 32
    gather_fn = _make_sc_gather(N, D, n_per, 128)
    gathered = gather_fn(table, idx_all)

    b2row = jnp.broadcast_to(b2.reshape(1, 1), (1, H))
    I = jnp.eye(H, dtype=jnp.float32)
    return _mlp_call(gathered, W1, b1.reshape(1, 64), W2, b2row, I, B, H, D)


# speedup vs baseline: 1.0171x; 1.0171x over previous
import functools
import jax
import jax.numpy as jnp
from jax import lax
from jax.experimental import pallas as pl
from jax.experimental.pallas import tpu as pltpu
from jax.experimental.pallas import tpu_sc as plsc


def _make_sc_gather(N, D, n_per, CH):
    info = plsc.get_sparse_core_info()
    NC = info.num_cores

    mesh = plsc.VectorSubcoreMesh(core_axis_name="c", subcore_axis_name="s")

    @functools.partial(
        pl.kernel,
        mesh=mesh,
        out_type=jax.ShapeDtypeStruct((N, 2 * D), jnp.float32),
        scratch_types=[
            pltpu.VMEM((n_per,), jnp.int32),
            pltpu.VMEM((n_per,), jnp.int32),
            pltpu.VMEM((3, 2 * CH, D), jnp.float32),
            pltpu.SemaphoreType.DMA,
            pltpu.SemaphoreType.DMA,
        ],
    )
    def gather_kernel(table_hbm, idx_hbm, out_hbm, idx1_v, idx2_v, stage_v,
                      gsem, wsem):
        c = lax.axis_index("c")
        s = lax.axis_index("s")
        wid = s * NC + c
        base = wid * n_per
        pltpu.sync_copy(idx_hbm.at[0, pl.ds(base, n_per)], idx1_v)
        pltpu.sync_copy(idx_hbm.at[1, pl.ds(base, n_per)], idx2_v)

        tasks = [(idx1_v, 0, r) for r in range(0, n_per, 2 * CH)]
        tasks += [(idx2_v, D, r) for r in range(0, n_per, 2 * CH)]
        S = 3
        gath = [None] * S
        writes = [None] * S

        def start_gather(k):
            sl = k % S
            idxv, _, roff = tasks[k]
            g1 = pltpu.async_copy(
                table_hbm.at[idxv.at[pl.ds(roff, CH)]],
                stage_v.at[sl, pl.ds(0, CH)], gsem,
            )
            g2 = pltpu.async_copy(
                table_hbm.at[idxv.at[pl.ds(roff + CH, CH)]],
                stage_v.at[sl, pl.ds(CH, CH)], gsem,
            )
            gath[sl] = (g1, g2)

        def start_write(k):
            sl = k % S
            _, coloff, roff = tasks[k]
            gath[sl][0].wait()
            gath[sl][1].wait()
            writes[sl] = pltpu.async_copy(
                stage_v.at[sl],
                out_hbm.at[pl.ds(base + roff, 2 * CH), pl.ds(coloff, D)],
                wsem,
            )

        nt = len(tasks)
        for k in range(nt):
            sl = k % S
            if writes[sl] is not None:
                writes[sl].wait()
                writes[sl] = None
            start_gather(k)
            if k >= 2:
                start_write(k - 2)
        for k in range(max(0, nt - 2), nt):
            start_write(k)
        for sl in range(S):
            if writes[sl] is not None:
                writes[sl].wait()

    return gather_kernel


_IB = 64


def _mlp_body(g_ref, w1_ref, b1_ref, w2_ref, b2_ref, i_ref, out_ref):
    H = i_ref.shape[0]
    f = g_ref[0, 0]
    t = jnp.dot(
        f.astype(jnp.bfloat16), w1_ref[...].astype(jnp.bfloat16),
        preferred_element_type=jnp.float32,
    ) + b1_ref[...]
    h = jnp.maximum(t, 0.0)
    o = jnp.dot(h, w2_ref[...], preferred_element_type=jnp.float32)
    rows = [
        lax.dot_general(
            o[r * H:(r + 1) * H], i_ref[...], (((0,), (0,)), ((), ())),
            preferred_element_type=jnp.float32,
        )
        for r in range(_IB)
    ]
    out_ref[0] = jnp.concatenate(rows, axis=0) + b2_ref[...]


def _mlp_call(gathered, W1, b1r, W2, b2row, I, B, H, D, interpret=False):
    n = _IB * H
    grid = (B, H // _IB)
    return pl.pallas_call(
        _mlp_body,
        grid=grid,
        in_specs=[
            pl.BlockSpec((1, 1, n, 2 * D), lambda b, i: (b, i, 0, 0)),
            pl.BlockSpec((2 * D, 64), lambda b, i: (0, 0)),
            pl.BlockSpec((1, 64), lambda b, i: (0, 0)),
            pl.BlockSpec((64, 1), lambda b, i: (0, 0)),
            pl.BlockSpec((1, H), lambda b, i: (0, 0)),
            pl.BlockSpec((H, H), lambda b, i: (0, 0)),
        ],
        out_specs=pl.BlockSpec((1, _IB, H), lambda b, i: (b, i, 0)),
        out_shape=jax.ShapeDtypeStruct((B, H, H), jnp.float32),
        interpret=interpret,
    )(gathered.reshape(B, H // _IB, n, 2 * D), W1, b1r, W2, b2row, I)


def kernel(pair_repr, helix_ends_list, W1, b1, W2, b2):
    B, L, _, D = pair_repr.shape
    H = helix_ends_list.shape[1]
    i5 = helix_ends_list[:, :, 1]
    i3 = helix_ends_list[:, :, 2]

    boff = (jnp.arange(B, dtype=jnp.int32) * (L * L))[:, None, None]
    idx1 = boff + i5[:, :, None] * L + i5[:, None, :]
    idx2 = boff + i3[:, :, None] * L + i3[:, None, :]
    idx_all = jnp.stack([idx1.reshape(-1), idx2.reshape(-1)], axis=0)

    table = pair_repr.reshape(B * L * L, D)
    N = B * H * H
    n_per = N // 32
    gather_fn = _make_sc_gather(N, D, n_per, 128)
    gathered = gather_fn(table, idx_all)

    b2row = jnp.broadcast_to(b2.reshape(1, 1), (1, H))
    I = jnp.eye(H, dtype=jnp.float32)
    return _mlp_call(gathered, W1, b1.reshape(1, 64), W2, b2row, I, B, H, D)
